# baseline (speedup 1.0000x reference)
import jax
import jax.numpy as jnp
from jax.experimental import pallas as pl

_EPS = 1e-6
_TOKEN_BLOCK = 256
_WINDOW = 2048


def _vq_rot_kernel(z_ref, zb_ref, eb_ref, emb_ref, a2_ref, b2_ref,
                   q_ref, idx_ref, acc_ref):
    i = pl.program_id(0)
    z = z_ref[...]
    zb = zb_ref[...]
    eb = eb_ref[...]
    emb = emb_ref[...]
    a2 = a2_ref[...]
    b2 = b2_ref[...]
    t, d = z.shape
    c = emb.shape[0]

    ab = jax.lax.dot_general(zb, eb, (((1,), (1,)), ((), ())),
                             preferred_element_type=jnp.float32)
    dist = (a2 - 2.0 * ab) + b2

    carry_v = jnp.full((t, 1), jnp.inf, jnp.float32)
    carry_i = jnp.zeros((t, 1), jnp.int32)
    for w in range(c // _WINDOW):
        dw = jax.lax.slice(dist, (0, w * _WINDOW), (t, (w + 1) * _WINDOW))
        m = jnp.min(dw, axis=1, keepdims=True)
        iota_w = jax.lax.broadcasted_iota(jnp.int32, (t, _WINDOW), 1)
        mi = jnp.min(jnp.where(dw == m, iota_w, _WINDOW), axis=1,
                     keepdims=True) + w * _WINDOW
        take = m < carry_v
        carry_v = jnp.where(take, m.astype(jnp.bfloat16).astype(jnp.float32),
                            carry_v)
        carry_i = jnp.where(take, mi, carry_i)
    idx_ref[...] = carry_i

    iota = jax.lax.broadcasted_iota(jnp.int32, (t, c), 1)
    onehot = (iota == carry_i).astype(jnp.float32)
    q = jax.lax.dot_general(onehot, emb, (((1,), (0,)), ((), ())),
                            precision=jax.lax.Precision.HIGHEST,
                            preferred_element_type=jnp.float32)

    z_norm = jnp.sqrt(jnp.sum(z * z, axis=1, keepdims=True))
    q_norm = jnp.sqrt(jnp.sum(q * q, axis=1, keepdims=True))
    z_hat = z / (z_norm + _EPS)
    q_hat = q / (q_norm + _EPS)
    v = z_hat - q_hat
    v = v / (jnp.sqrt(jnp.sum(v * v, axis=1, keepdims=True)) + _EPS)
    rz = z - 2.0 * jnp.sum(v * z, axis=1, keepdims=True) * v
    s = q_norm / (z_norm + _EPS)
    q_ref[...] = s * rz

    diff = z - q
    part = jnp.sum(diff * diff).reshape(1, 1)

    @pl.when(i == 0)
    def _():
        acc_ref[...] = jnp.zeros((1, 1), jnp.float32)

    acc_ref[...] += part


@jax.jit
def kernel(z_e, embedding):
    b, d, h, w = z_e.shape
    c = embedding.shape[0]
    n = b * h * w
    t = _TOKEN_BLOCK
    z_flat = jnp.transpose(z_e, (0, 2, 3, 1)).reshape(n, d)
    z_bf = z_flat.astype(jnp.bfloat16)
    e_bf = embedding.astype(jnp.bfloat16)
    a2 = jnp.sum(z_flat ** 2, axis=1, keepdims=True)
    b2 = jnp.sum(embedding ** 2, axis=1).reshape(1, c)

    q_tilde, idx, acc = pl.pallas_call(
        _vq_rot_kernel,
        grid=(n // t,),
        in_specs=[
            pl.BlockSpec((t, d), lambda i: (i, 0)),
            pl.BlockSpec((t, d), lambda i: (i, 0)),
            pl.BlockSpec((c, d), lambda i: (0, 0)),
            pl.BlockSpec((c, d), lambda i: (0, 0)),
            pl.BlockSpec((t, 1), lambda i: (i, 0)),
            pl.BlockSpec((1, c), lambda i: (0, 0)),
        ],
        out_specs=[
            pl.BlockSpec((t, d), lambda i: (i, 0)),
            pl.BlockSpec((t, 1), lambda i: (i, 0)),
            pl.BlockSpec((1, 1), lambda i: (0, 0)),
        ],
        out_shape=[
            jax.ShapeDtypeStruct((n, d), jnp.float32),
            jax.ShapeDtypeStruct((n, 1), jnp.int32),
            jax.ShapeDtypeStruct((1, 1), jnp.float32),
        ],
    )(z_flat, z_bf, e_bf, embedding, a2, b2)

    z_q = jnp.transpose(q_tilde.reshape(b, h, w, d), (0, 3, 1, 2))
    indices_out = idx.reshape(b, h, w)
    commit_loss = (0.25 / (n * d)) * acc[0, 0]
    return (z_q, indices_out, commit_loss)


# speedup vs baseline: 1.8033x; 1.8033x over previous
import jax
import jax.numpy as jnp
from jax.experimental import pallas as pl

_EPS = 1e-6
_TOKEN_BLOCK = 256
_WINDOW = 2048


def _vq_rot_kernel(z_ref, zb_ref, eb_ref, emb_ref, a2_ref, b2_ref,
                   q_ref, idx_ref, acc_ref):
    i = pl.program_id(0)
    z = z_ref[...]
    zb = zb_ref[...]
    eb = eb_ref[...]
    emb = emb_ref[...]
    a2 = a2_ref[...]
    b2 = b2_ref[...]
    t, d = z.shape
    c = emb.shape[0]

    ab = jax.lax.dot_general(zb, eb, (((1,), (1,)), ((), ())),
                             preferred_element_type=jnp.float32)
    dist = (a2 - 2.0 * ab) + b2

    carry_v = jnp.full((t, 1), jnp.inf, jnp.float32)
    carry_i = jnp.zeros((t, 1), jnp.int32)
    for w in range(c // _WINDOW):
        dw = jax.lax.slice(dist, (0, w * _WINDOW), (t, (w + 1) * _WINDOW))
        m = jnp.min(dw, axis=1, keepdims=True)
        iota_w = jax.lax.broadcasted_iota(jnp.int32, (t, _WINDOW), 1)
        mi = jnp.min(jnp.where(dw == m, iota_w, _WINDOW), axis=1,
                     keepdims=True) + w * _WINDOW
        take = m < carry_v
        carry_v = jnp.where(take, m.astype(jnp.bfloat16).astype(jnp.float32),
                            carry_v)
        carry_i = jnp.where(take, mi, carry_i)
    idx_ref[...] = carry_i

    iota = jax.lax.broadcasted_iota(jnp.int32, (t, c), 1)
    onehot = (iota == carry_i).astype(jnp.bfloat16)
    e_lo = (emb - eb.astype(jnp.float32)).astype(jnp.bfloat16)
    q_hi = jax.lax.dot_general(onehot, eb, (((1,), (0,)), ((), ())),
                               preferred_element_type=jnp.float32)
    q_lo = jax.lax.dot_general(onehot, e_lo, (((1,), (0,)), ((), ())),
                               preferred_element_type=jnp.float32)
    q = q_hi + q_lo

    z_norm = jnp.sqrt(jnp.sum(z * z, axis=1, keepdims=True))
    q_norm = jnp.sqrt(jnp.sum(q * q, axis=1, keepdims=True))
    z_hat = z / (z_norm + _EPS)
    q_hat = q / (q_norm + _EPS)
    v = z_hat - q_hat
    v = v / (jnp.sqrt(jnp.sum(v * v, axis=1, keepdims=True)) + _EPS)
    rz = z - 2.0 * jnp.sum(v * z, axis=1, keepdims=True) * v
    s = q_norm / (z_norm + _EPS)
    q_ref[...] = s * rz

    diff = z - q
    part = jnp.sum(diff * diff).reshape(1, 1)

    @pl.when(i == 0)
    def _():
        acc_ref[...] = jnp.zeros((1, 1), jnp.float32)

    acc_ref[...] += part


@jax.jit
def kernel(z_e, embedding):
    b, d, h, w = z_e.shape
    c = embedding.shape[0]
    n = b * h * w
    t = _TOKEN_BLOCK
    z_flat = jnp.transpose(z_e, (0, 2, 3, 1)).reshape(n, d)
    z_bf = z_flat.astype(jnp.bfloat16)
    e_bf = embedding.astype(jnp.bfloat16)
    a2 = jnp.sum(z_flat ** 2, axis=1, keepdims=True)
    b2 = jnp.sum(embedding ** 2, axis=1).reshape(1, c)

    q_tilde, idx, acc = pl.pallas_call(
        _vq_rot_kernel,
        grid=(n // t,),
        in_specs=[
            pl.BlockSpec((t, d), lambda i: (i, 0)),
            pl.BlockSpec((t, d), lambda i: (i, 0)),
            pl.BlockSpec((c, d), lambda i: (0, 0)),
            pl.BlockSpec((c, d), lambda i: (0, 0)),
            pl.BlockSpec((t, 1), lambda i: (i, 0)),
            pl.BlockSpec((1, c), lambda i: (0, 0)),
        ],
        out_specs=[
            pl.BlockSpec((t, d), lambda i: (i, 0)),
            pl.BlockSpec((t, 1), lambda i: (i, 0)),
            pl.BlockSpec((1, 1), lambda i: (0, 0)),
        ],
        out_shape=[
            jax.ShapeDtypeStruct((n, d), jnp.float32),
            jax.ShapeDtypeStruct((n, 1), jnp.int32),
            jax.ShapeDtypeStruct((1, 1), jnp.float32),
        ],
    )(z_flat, z_bf, e_bf, embedding, a2, b2)

    z_q = jnp.transpose(q_tilde.reshape(b, h, w, d), (0, 3, 1, 2))
    indices_out = idx.reshape(b, h, w)
    commit_loss = (0.25 / (n * d)) * acc[0, 0]
    return (z_q, indices_out, commit_loss)
